# Initial kernel scaffold; baseline (speedup 1.0000x reference)
import jax
import jax.numpy as jnp
from jax.experimental import pallas as pl


def kernel(x, tables):
    raise NotImplementedError("write your pallas kernel here")


# speedup vs baseline: 42.5616x; 42.5616x over previous
import math

import jax
import jax.numpy as jnp
import numpy as np
from jax import lax
from jax.experimental import pallas as pl
from jax.experimental.pallas import tpu as pltpu
from jax.experimental.pallas import tpu_sc as plsc

NUM_LEVELS = 16
MIN_RES = 16
MAX_RES = 2048
EMB_DIM = 4
HASHMAP = 524288
B = 1048576

_RES = np.round(
    np.logspace(math.log10(float(MIN_RES)), math.log10(float(MAX_RES)), NUM_LEVELS)
).astype(np.int32)
_OFFS = np.concatenate([[0], np.cumsum(_RES)[:-1]]).astype(np.int32)
R_TOTAL = int(_RES.sum())

def _compact_word_indices() -> np.ndarray:
    parts = []
    for lvl in range(NUM_LEVELS):
        r = int(_RES[lvl])
        i = np.arange(r, dtype=np.int64)
        h = ((i * 73856093) ^ (lvl * 19349663)) & (HASHMAP - 1)
        rows = lvl * HASHMAP + h
        words = (rows[:, None] * EMB_DIM + np.arange(EMB_DIM)[None, :]).reshape(-1)
        parts.append(words)
    return np.concatenate(parts)


_GCHUNK = 128
_GBATCH = 8
W_TOTAL = R_TOTAL * EMB_DIM
W_PAD = ((W_TOTAL + _GCHUNK * _GBATCH - 1) // (_GCHUNK * _GBATCH)) * (_GCHUNK * _GBATCH)
_CIDX = np.zeros((W_PAD,), dtype=np.int32)
_CIDX[:W_TOTAL] = _compact_word_indices()

NC, NS = 2, 16
NW = NC * NS
PT = B // NW
CHUNK = 512
NGRP = CHUNK // 16


def _body(tab_hbm, cidx_hbm, x_hbm, out_hbm, cidx_v, compact_v, x_v, out_v, sem):
    cid = lax.axis_index("c")
    sid = lax.axis_index("s")
    wid = sid * NC + cid

    pltpu.sync_copy(cidx_hbm, cidx_v)

    def gather_step(j, carry):
        copies = []
        for b in range(_GBATCH):
            o = (j * _GBATCH + b) * _GCHUNK
            copies.append(
                pltpu.async_copy(
                    tab_hbm.at[cidx_v.at[pl.ds(o, _GCHUNK)]],
                    compact_v.at[pl.ds(o, _GCHUNK)],
                    sem,
                )
            )
        for cp in copies:
            cp.wait()
        return carry

    lax.fori_loop(0, W_PAD // (_GCHUNK * _GBATCH), gather_step, 0)

    iota = lax.iota(jnp.int32, 16)
    base_pt = wid * PT

    def chunk_body(c, carry):
        pb = base_pt + c * CHUNK
        pltpu.sync_copy(x_hbm.at[pl.ds(pb, CHUNK)], x_v)

        def grp_body(g, carry2):
            xv = x_v[pl.ds(g * 16, 16)]
            xc = jnp.minimum(jnp.maximum(xv, jnp.float32(0.0)), jnp.float32(1.0))
            rowbase = (g * 16 + iota) * 64
            for lvl in range(NUM_LEVELS):
                rl = int(_RES[lvl])
                off4 = int(_OFFS[lvl]) * EMB_DIM
                t = xc * jnp.float32(rl - 1)
                i0 = t.astype(jnp.int32)
                w = t - i0.astype(jnp.float32)
                omw = jnp.float32(1.0) - w
                i1 = jnp.minimum(i0 + 1, rl - 1)
                f0 = i0 * EMB_DIM
                f1 = i1 * EMB_DIM
                for d in range(EMB_DIM):
                    e0 = plsc.load_gather(compact_v, [f0 + (off4 + d)])
                    e1 = plsc.load_gather(compact_v, [f1 + (off4 + d)])
                    plsc.store_scatter(
                        out_v, [rowbase + (lvl * EMB_DIM + d)], e0 * omw + e1 * w
                    )
            return carry2

        lax.fori_loop(0, NGRP, grp_body, 0)
        pltpu.sync_copy(out_v, out_hbm.at[pl.ds(pb * 64, CHUNK * 64)])
        return carry

    lax.fori_loop(0, PT // CHUNK, chunk_body, 0)


_mesh = plsc.VectorSubcoreMesh(core_axis_name="c", subcore_axis_name="s")

_sc_call = pl.kernel(
    _body,
    out_type=jax.ShapeDtypeStruct((B * NUM_LEVELS * EMB_DIM,), jnp.float32),
    mesh=_mesh,
    compiler_params=pltpu.CompilerParams(needs_layout_passes=False),
    scratch_types=[
        pltpu.VMEM((W_PAD,), jnp.int32),
        pltpu.VMEM((W_PAD,), jnp.float32),
        pltpu.VMEM((CHUNK,), jnp.float32),
        pltpu.VMEM((CHUNK * NUM_LEVELS * EMB_DIM,), jnp.float32),
        pltpu.SemaphoreType.DMA,
    ],
)


def kernel(x, tables):
    tab_flat = tables.reshape(NUM_LEVELS * HASHMAP * EMB_DIM)
    cidx = jnp.asarray(_CIDX)
    out_flat = _sc_call(tab_flat, cidx, x)
    return out_flat.reshape(B, NUM_LEVELS * EMB_DIM)
